# baseline (speedup 1.0000x reference)
import jax
import jax.numpy as jnp
from jax.experimental import pallas as pl
from jax.experimental.pallas import tpu as pltpu


_NCHUNK = 16


def _body(ctx_ref, rank_ref, sent_ref, out_ref, sems):
    num_ranks = out_ref.shape[0]
    cb = num_ranks // _NCHUNK
    copies = []
    for i in range(_NCHUNK):
        sl = pl.ds(i * cb, cb)
        copies.append(pltpu.make_async_copy(
            sent_ref.at[sl], out_ref.at[sl], sems.at[i]))
    for cp in copies:
        cp.start()
    for cp in copies:
        cp.wait()


def kernel(context_embeds, rank_embeds, sentence_embeds):
    num_ranks, max_tokens, dim = sentence_embeds.shape
    return pl.pallas_call(
        _body,
        in_specs=[
            pl.BlockSpec(memory_space=pltpu.VMEM),
            pl.BlockSpec(memory_space=pl.ANY),
            pl.BlockSpec(memory_space=pl.ANY),
        ],
        out_specs=pl.BlockSpec(memory_space=pl.ANY),
        out_shape=jax.ShapeDtypeStruct(
            (num_ranks, max_tokens, dim), sentence_embeds.dtype),
        scratch_shapes=[pltpu.SemaphoreType.DMA((_NCHUNK,))],
    )(context_embeds, rank_embeds, sentence_embeds)


# speedup vs baseline: 15.6311x; 15.6311x over previous
import jax
import jax.numpy as jnp
from jax.experimental import pallas as pl
from jax.experimental.pallas import tpu as pltpu


_NCHUNK = 32
_NBUF = 8
_LOOK = 4


def _body(ctx_ref, rank_ref, sent_ref, out_ref, bufs, in_sems, out_sems):
    num_ranks, max_tokens, dim = out_ref.shape
    c = ctx_ref.shape[0]
    k = rank_ref.shape[1]
    cb = num_ranks // _NCHUNK

    def reads(j):
        b = j % _NBUF
        sl = pl.ds(j * cb, cb)
        return (
            pltpu.make_async_copy(
                sent_ref.at[sl, pl.ds(0, 1)], bufs.at[b, :, pl.ds(0, 1)],
                in_sems.at[j, 0]),
            pltpu.make_async_copy(
                sent_ref.at[sl, pl.ds(c, max_tokens - c)],
                bufs.at[b, :, pl.ds(c, max_tokens - c)],
                in_sems.at[j, 1]),
        )

    def write(j):
        b = j % _NBUF
        sl = pl.ds(j * cb, cb)
        return pltpu.make_async_copy(bufs.at[b], out_ref.at[sl], out_sems.at[j])

    def process(j):
        b = j % _NBUF
        for cp in reads(j):
            cp.wait()
        bufs[b, :, 1:1 + c, :] = jnp.broadcast_to(
            ctx_ref[...][None], (cb, c, dim))
        bufs[b, :, 1 + c:1 + c + k, :] = rank_ref[pl.ds(j * cb, cb)]
        write(j).start()

    for j in range(_NCHUNK):
        if j >= _NBUF:
            write(j - _NBUF).wait()
        for cp in reads(j):
            cp.start()
        p = j - _LOOK
        if p >= 0:
            process(p)
    for p in range(_NCHUNK - _LOOK, _NCHUNK):
        process(p)
    for j in range(_NCHUNK - _NBUF, _NCHUNK):
        write(j).wait()


def kernel(context_embeds, rank_embeds, sentence_embeds):
    num_ranks, max_tokens, dim = sentence_embeds.shape
    c, _ = context_embeds.shape
    k = rank_embeds.shape[1]
    cb = num_ranks // _NCHUNK
    dt = sentence_embeds.dtype
    return pl.pallas_call(
        _body,
        in_specs=[
            pl.BlockSpec(memory_space=pltpu.VMEM),
            pl.BlockSpec(memory_space=pltpu.VMEM),
            pl.BlockSpec(memory_space=pl.ANY),
        ],
        out_specs=pl.BlockSpec(memory_space=pl.ANY),
        out_shape=jax.ShapeDtypeStruct((num_ranks, max_tokens, dim), dt),
        scratch_shapes=[
            pltpu.VMEM((_NBUF, cb, max_tokens, dim), dt),
            pltpu.SemaphoreType.DMA((_NCHUNK, 2)),
            pltpu.SemaphoreType.DMA((_NCHUNK,)),
        ],
    )(context_embeds, rank_embeds, sentence_embeds)
